# Initial kernel scaffold; baseline (speedup 1.0000x reference)
import jax
import jax.numpy as jnp
from jax.experimental import pallas as pl


def kernel(x, adj, W1, b1, W2, b2):
    raise NotImplementedError("write your pallas kernel here")


# speedup vs baseline: 11.0987x; 11.0987x over previous
import functools

import jax
import jax.numpy as jnp
from jax import lax
from jax.experimental import pallas as pl
from jax.experimental.pallas import tpu as pltpu
from jax.experimental.pallas import tpu_sc as plsc

N = 10000
NP = 10240
D = 128
E = 320000
EP = 323584
EDGES_PER_TILE = EP // 32
CHUNK = 128
NCHUNKS = EDGES_PER_TILE // CHUNK
ROWS_PER_TILE = NP // 16
TRASH = N

_MESH = plsc.VectorSubcoreMesh(core_axis_name="c", subcore_axis_name="s")


def _sc_deg_body(dst_hbm, zer_hbm, ones_hbm, out_hbm, onesv, didx, acc, sem):
    c = lax.axis_index("c")
    s = lax.axis_index("s")
    pltpu.sync_copy(ones_hbm, onesv)
    pltpu.sync_copy(zer_hbm, acc.at[pl.ds(s * ROWS_PER_TILE, ROWS_PER_TILE)])
    plsc.subcore_barrier()
    base = (c * 16 + s) * EDGES_PER_TILE

    def chunk(i, carry):
        pltpu.sync_copy(dst_hbm.at[pl.ds(base + i * CHUNK, CHUNK)], didx)
        pltpu.sync_copy(onesv, acc.at[didx], add=True)
        return carry

    lax.fori_loop(0, NCHUNKS, chunk, 0)
    plsc.subcore_barrier()
    r0 = s * ROWS_PER_TILE
    pltpu.sync_copy(acc.at[pl.ds(r0, ROWS_PER_TILE)],
                    out_hbm.at[c, pl.ds(r0, ROWS_PER_TILE)])


_sc_deg = functools.partial(
    pl.kernel,
    out_type=jax.ShapeDtypeStruct((2, NP, D), jnp.float32),
    mesh=_MESH,
    scratch_types=[
        pltpu.VMEM((CHUNK, D), jnp.float32),
        pltpu.VMEM((CHUNK,), jnp.int32),
        pltpu.VMEM_SHARED((NP, D), jnp.float32),
        pltpu.SemaphoreType.DMA,
    ],
)(_sc_deg_body)


def _sc_gs_body(h_hbm, src_hbm, dst_hbm, zer_hbm, out_hbm,
                sidx, didx, rows, acc, sem):
    c = lax.axis_index("c")
    s = lax.axis_index("s")
    pltpu.sync_copy(zer_hbm, acc.at[pl.ds(s * ROWS_PER_TILE, ROWS_PER_TILE)])
    plsc.subcore_barrier()
    base = (c * 16 + s) * EDGES_PER_TILE

    def chunk(i, carry):
        off = base + i * CHUNK
        pltpu.sync_copy(src_hbm.at[pl.ds(off, CHUNK)], sidx)
        pltpu.sync_copy(dst_hbm.at[pl.ds(off, CHUNK)], didx)
        pltpu.async_copy(h_hbm.at[sidx], rows, sem).wait()
        pltpu.sync_copy(rows, acc.at[didx], add=True)
        return carry

    lax.fori_loop(0, NCHUNKS, chunk, 0)
    plsc.subcore_barrier()
    r0 = s * ROWS_PER_TILE
    pltpu.sync_copy(acc.at[pl.ds(r0, ROWS_PER_TILE)],
                    out_hbm.at[c, pl.ds(r0, ROWS_PER_TILE)])


_sc_gs = functools.partial(
    pl.kernel,
    out_type=jax.ShapeDtypeStruct((2, NP, D), jnp.float32),
    mesh=_MESH,
    scratch_types=[
        pltpu.VMEM((CHUNK,), jnp.int32),
        pltpu.VMEM((CHUNK,), jnp.int32),
        pltpu.VMEM((CHUNK, D), jnp.float32),
        pltpu.VMEM_SHARED((NP, D), jnp.float32),
        pltpu.SemaphoreType.DMA,
    ],
)(_sc_gs_body)


_BLK = 1024
_GRID = NP // _BLK


def _dinv_block(dp_ref):
    deg = dp_ref[0, :, 0:1] + dp_ref[1, :, 0:1] + 1.0
    return lax.rsqrt(deg)


def _tc_h1_body(x_ref, w_ref, dp_ref, o_ref):
    dinv = _dinv_block(dp_ref)
    o_ref[:, :] = dinv * jnp.dot(x_ref[:, :], w_ref[:, :],
                                 preferred_element_type=jnp.float32)


def _tc_h1(xp, w1t, dparts):
    return pl.pallas_call(
        _tc_h1_body,
        grid=(_GRID,),
        in_specs=[
            pl.BlockSpec((_BLK, D), lambda i: (i, 0)),
            pl.BlockSpec((D, D), lambda i: (0, 0)),
            pl.BlockSpec((2, _BLK, D), lambda i: (0, i, 0)),
        ],
        out_specs=pl.BlockSpec((_BLK, D), lambda i: (i, 0)),
        out_shape=jax.ShapeDtypeStruct((NP, D), jnp.float32),
    )(xp, w1t, dparts)


def _tc_mid_body(s_ref, h_ref, dp_ref, b_ref, w_ref, o_ref):
    dinv = _dinv_block(dp_ref)
    z = dinv * (s_ref[0, :, :] + s_ref[1, :, :] + h_ref[:, :]) + b_ref[:, :]
    a = jnp.maximum(z, 0.0)
    o_ref[:, :] = dinv * jnp.dot(a, w_ref[:, :],
                                 preferred_element_type=jnp.float32)


def _tc_mid(s1, h1p, dparts, b1r, w2t):
    return pl.pallas_call(
        _tc_mid_body,
        grid=(_GRID,),
        in_specs=[
            pl.BlockSpec((2, _BLK, D), lambda i: (0, i, 0)),
            pl.BlockSpec((_BLK, D), lambda i: (i, 0)),
            pl.BlockSpec((2, _BLK, D), lambda i: (0, i, 0)),
            pl.BlockSpec((1, D), lambda i: (0, 0)),
            pl.BlockSpec((D, D), lambda i: (0, 0)),
        ],
        out_specs=pl.BlockSpec((_BLK, D), lambda i: (i, 0)),
        out_shape=jax.ShapeDtypeStruct((NP, D), jnp.float32),
    )(s1, h1p, dparts, b1r, w2t)


def _tc_out_body(s_ref, h_ref, dp_ref, b_ref, o_ref):
    dinv = _dinv_block(dp_ref)
    o_ref[:, :] = dinv * (s_ref[0, :, :] + s_ref[1, :, :] + h_ref[:, :]) \
        + b_ref[:, :]


def _tc_out(s2, h2p, dparts, b2r):
    return pl.pallas_call(
        _tc_out_body,
        grid=(_GRID,),
        in_specs=[
            pl.BlockSpec((2, _BLK, D), lambda i: (0, i, 0)),
            pl.BlockSpec((_BLK, D), lambda i: (i, 0)),
            pl.BlockSpec((2, _BLK, D), lambda i: (0, i, 0)),
            pl.BlockSpec((1, D), lambda i: (0, 0)),
        ],
        out_specs=pl.BlockSpec((_BLK, D), lambda i: (i, 0)),
        out_shape=jax.ShapeDtypeStruct((NP, D), jnp.float32),
    )(s2, h2p, dparts, b2r)


def kernel(x, adj, W1, b1, W2, b2):
    src = adj[0].astype(jnp.int32)
    dst = adj[1].astype(jnp.int32)
    pad = EP - E
    srcp = jnp.concatenate([src, jnp.zeros((pad,), jnp.int32)])
    dstp = jnp.concatenate([dst, jnp.full((pad,), TRASH, jnp.int32)])
    xp = jnp.pad(x, ((0, NP - N), (0, 0)))
    w1t = W1.T
    w2t = W2.T
    b1r = b1.reshape(1, D)
    b2r = b2.reshape(1, D)
    zer = jnp.zeros((ROWS_PER_TILE, D), jnp.float32)
    ones128 = jnp.ones((CHUNK, D), jnp.float32)

    dparts = _sc_deg(dstp, zer, ones128)
    h1p = _tc_h1(xp, w1t, dparts)
    s1 = _sc_gs(h1p, srcp, dstp, zer)
    h2p = _tc_mid(s1, h1p, dparts, b1r, w2t)
    s2 = _sc_gs(h2p, srcp, dstp, zer)
    outp = _tc_out(s2, h2p, dparts, b2r)
    return outp[:N]
